# baseline (speedup 1.0000x reference)
import functools

import jax
import jax.numpy as jnp
from jax import lax
from jax.experimental import pallas as pl
from jax.experimental.pallas import tpu as pltpu
from jax.experimental.pallas import tpu_sc as plsc

R = 64
N = 8192
KK = 64
D = 128
L = 16
NCHUNK = 64
CHUNK = 128
NVC = CHUNK // L

_info = plsc.get_sparse_core_info()
NC, NS = _info.num_cores, _info.num_subcores
NW = NC * NS
ROWS_PER_W = R // NW

_NEG = float("-inf")
_BIG = 1 << 20

_mesh = plsc.VectorSubcoreMesh(core_axis_name="c", subcore_axis_name="s")


@functools.partial(
    pl.kernel,
    mesh=_mesh,
    out_type=[
        jax.ShapeDtypeStruct((R, KK), jnp.float32),
        jax.ShapeDtypeStruct((R, KK), jnp.float32),
        jax.ShapeDtypeStruct((R, KK, D), jnp.float32),
    ],
    scratch_types=[
        pltpu.VMEM((2 * N,), jnp.float32),
        pltpu.VMEM((2 * N,), jnp.float32),
        [pltpu.VMEM((KK,), jnp.float32) for _ in range(2)],
        [pltpu.VMEM((KK,), jnp.float32) for _ in range(2)],
        pltpu.VMEM((2 * KK,), jnp.int32),
        [pltpu.VMEM((KK,), jnp.int32) for _ in range(2)],
        [pltpu.VMEM((KK, D), jnp.float32) for _ in range(2)],
        pltpu.SemaphoreType.DMA,
        pltpu.SemaphoreType.DMA,
        pltpu.SemaphoreType.DMA,
    ],
    compiler_params=pltpu.CompilerParams(needs_layout_passes=False),
)
def _topk_gather(w_hbm, p1_hbm, p2_hbm, outw, outg1, outg2,
                 row_v, p1row_v, vals_vs, g1_vs, idx_v, gidx_vs,
                 rows_vs, sem_w, sem_p1, sem_g2):
    wid = lax.axis_index("s") * NC + lax.axis_index("c")
    rowA = wid * ROWS_PER_W
    iota = lax.iota(jnp.int32, L)
    lane0 = iota == 0

    def bfly(x, op):
        for s in (1, 2, 4, 8):
            x = op(x, x.at[jnp.bitwise_xor(iota, s)].get(
                mode="promise_in_bounds"))
        return x

    def bfly_argmax(v, i):
        for s in (1, 2, 4, 8):
            perm = jnp.bitwise_xor(iota, s)
            pv = v.at[perm].get(mode="promise_in_bounds")
            pi = i.at[perm].get(mode="promise_in_bounds")
            sw = (pv > v) | ((pv == v) & (pi < i))
            v = jnp.where(sw, pv, v)
            i = jnp.where(sw, pi, i)
        return v, i

    def store1(ref, pos_v, val_v):
        plsc.store_scatter(ref, [pos_v], val_v, mask=lane0)

    cps = [pltpu.async_copy(w_hbm.at[rowA + r], row_v.at[pl.ds(r * N, N)],
                            sem_w) for r in range(2)]
    cp1s = [pltpu.async_copy(p1_hbm.at[rowA + r],
                             p1row_v.at[pl.ds(r * N, N)], sem_p1)
            for r in range(2)]
    for cp in cps:
        cp.wait()

    base_idx = [[(iota + 16 * q) * CHUNK + r * N for q in range(4)]
                for r in range(2)]
    TU = 4

    def cm_step(t, carry):
        accs = list(carry)
        for u in range(TU):
            ph = (t * TU + u + iota) & (CHUNK - 1)
            p = 0
            for r in range(2):
                for q in range(4):
                    g = plsc.load_gather(row_v, [base_idx[r][q] + ph])
                    accs[p] = jnp.maximum(accs[p], g)
                    p += 1
        return tuple(accs)

    neg = jnp.full((L,), _NEG, jnp.float32)
    carry0 = lax.fori_loop(0, CHUNK // TU, cm_step, (neg,) * 8)

    def step(k, carry):
        out = []
        k_v = jnp.full((L,), k, jnp.int32)
        for r in range(2):
            c0, c1, c2, c3 = carry[4 * r:4 * r + 4]
            sa = c1 > c0
            va = jnp.where(sa, c1, c0)
            ia = jnp.where(sa, iota + L, iota)
            sb = c3 > c2
            vb = jnp.where(sb, c3, c2)
            ib = jnp.where(sb, iota + 3 * L, iota + 2 * L)
            sc = vb > va
            v4 = jnp.where(sc, vb, va)
            i4 = jnp.where(sc, ib, ia)
            M, cstar = bfly_argmax(v4, i4)
            base_v = cstar * CHUNK + (r * N + iota)
            vs = [plsc.load_gather(row_v, [base_v + j * L])
                  for j in range(NVC)]
            cand = jnp.where(vs[0] == M, iota, _BIG)
            for j in range(1, NVC):
                cand = jnp.minimum(cand,
                                   jnp.where(vs[j] == M, iota + j * L, _BIG))
            eloc = bfly(cand, jnp.minimum)
            ei = cstar * CHUNK + eloc
            store1(vals_vs[r], k_v, M)
            store1(idx_v, k_v + r * KK, ei)
            mm = jnp.where(iota == eloc, _NEG, vs[0])
            for j in range(1, NVC):
                mm = jnp.maximum(mm, jnp.where(iota + j * L == eloc,
                                               _NEG, vs[j]))
            newmax = bfly(mm, jnp.maximum)
            store1(row_v, ei + r * N, jnp.full((L,), _NEG, jnp.float32))
            for q in range(4):
                out.append(jnp.where(iota + q * L == cstar, newmax,
                                     carry[4 * r + q]))
        return tuple(out)
    lax.fori_loop(0, KK, step, carry0, unroll=2)

    for cp in cp1s:
        cp.wait()
    for r in range(2):
        for t in range(KK // L):
            iv = idx_v[pl.ds(r * KK + t * L, L)]
            g1_vs[r][pl.ds(t * L, L)] = plsc.load_gather(p1row_v, [iv + r * N])
            gidx_vs[r][pl.ds(t * L, L)] = iv + (rowA + r) * N
    cpgs = [pltpu.async_copy(p2_hbm.at[gidx_vs[r]], rows_vs[r], sem_g2)
            for r in range(2)]
    for r in range(2):
        pltpu.sync_copy(vals_vs[r], outw.at[rowA + r])
        pltpu.sync_copy(g1_vs[r], outg1.at[rowA + r])
    for r in range(2):
        cpgs[r].wait()
        pltpu.sync_copy(rows_vs[r], outg2.at[rowA + r])


def kernel(weights, prop1, prop2):
    p2 = prop2.reshape(R * N, D)
    outw, outg1, outg2 = _topk_gather(weights, prop1, p2)
    return (outw, outg1, outg2)


# speedup vs baseline: 1.0096x; 1.0096x over previous
import functools

import jax
import jax.numpy as jnp
from jax import lax
from jax.experimental import pallas as pl
from jax.experimental.pallas import tpu as pltpu
from jax.experimental.pallas import tpu_sc as plsc

R = 64
N = 8192
KK = 64
D = 128
L = 16
NCHUNK = 64
CHUNK = 128
NVC = CHUNK // L

_info = plsc.get_sparse_core_info()
NC, NS = _info.num_cores, _info.num_subcores
NW = NC * NS
ROWS_PER_W = R // NW

_NEG = float("-inf")
_BIG = 1 << 20

_mesh = plsc.VectorSubcoreMesh(core_axis_name="c", subcore_axis_name="s")


@functools.partial(
    pl.kernel,
    mesh=_mesh,
    out_type=[
        jax.ShapeDtypeStruct((R, KK), jnp.float32),
        jax.ShapeDtypeStruct((R, KK), jnp.float32),
        jax.ShapeDtypeStruct((R, KK, D), jnp.float32),
    ],
    scratch_types=[
        pltpu.VMEM((2 * N,), jnp.float32),
        pltpu.VMEM((2 * N,), jnp.float32),
        [pltpu.VMEM((KK,), jnp.float32) for _ in range(2)],
        [pltpu.VMEM((KK,), jnp.float32) for _ in range(2)],
        pltpu.VMEM((2 * KK,), jnp.int32),
        [pltpu.VMEM((KK,), jnp.int32) for _ in range(2)],
        [pltpu.VMEM((KK, D), jnp.float32) for _ in range(2)],
        pltpu.SemaphoreType.DMA,
        pltpu.SemaphoreType.DMA,
        pltpu.SemaphoreType.DMA,
    ],
    compiler_params=pltpu.CompilerParams(needs_layout_passes=False),
)
def _topk_gather(w_hbm, p1_hbm, p2_hbm, outw, outg1, outg2,
                 row_v, p1row_v, vals_vs, g1_vs, idx_v, gidx_vs,
                 rows_vs, sem_w, sem_p1, sem_g2):
    wid = lax.axis_index("s") * NC + lax.axis_index("c")
    rowA = wid * ROWS_PER_W
    iota = lax.iota(jnp.int32, L)
    lane0 = iota == 0

    def bfly(x, op):
        for s in (1, 2, 4, 8):
            x = op(x, x.at[jnp.bitwise_xor(iota, s)].get(
                mode="promise_in_bounds"))
        return x

    def bfly_argmax(v, i):
        for s in (1, 2, 4, 8):
            perm = jnp.bitwise_xor(iota, s)
            pv = v.at[perm].get(mode="promise_in_bounds")
            pi = i.at[perm].get(mode="promise_in_bounds")
            sw = (pv > v) | ((pv == v) & (pi < i))
            v = jnp.where(sw, pv, v)
            i = jnp.where(sw, pi, i)
        return v, i

    def store1(ref, pos_v, val_v):
        plsc.store_scatter(ref, [pos_v], val_v, mask=lane0)

    cps = [pltpu.async_copy(w_hbm.at[rowA + r], row_v.at[pl.ds(r * N, N)],
                            sem_w) for r in range(2)]
    cp1s = [pltpu.async_copy(p1_hbm.at[rowA + r],
                             p1row_v.at[pl.ds(r * N, N)], sem_p1)
            for r in range(2)]
    for cp in cps:
        cp.wait()

    base_idx = [[(iota + 16 * q) * CHUNK + r * N for q in range(4)]
                for r in range(2)]
    TU = 4

    def cm_step(t, carry):
        accs = list(carry)
        for u in range(TU):
            ph = (t * TU + u + iota) & (CHUNK - 1)
            p = 0
            for r in range(2):
                for q in range(4):
                    g = plsc.load_gather(row_v, [base_idx[r][q] + ph])
                    accs[p] = jnp.maximum(accs[p], g)
                    p += 1
        return tuple(accs)

    neg = jnp.full((L,), _NEG, jnp.float32)
    carry0 = lax.fori_loop(0, CHUNK // TU, cm_step, (neg,) * 8)

    def step(k, carry):
        out = []
        k_v = jnp.full((L,), k, jnp.int32)
        for r in range(2):
            c0, c1, c2, c3 = carry[4 * r:4 * r + 4]
            sa = c1 > c0
            va = jnp.where(sa, c1, c0)
            ia = jnp.where(sa, iota + L, iota)
            sb = c3 > c2
            vb = jnp.where(sb, c3, c2)
            ib = jnp.where(sb, iota + 3 * L, iota + 2 * L)
            sc = vb > va
            v4 = jnp.where(sc, vb, va)
            i4 = jnp.where(sc, ib, ia)
            M = bfly(v4, jnp.maximum)
            cstar = bfly(jnp.where(v4 == M, i4, _BIG), jnp.minimum)
            base_v = cstar * CHUNK + (r * N + iota)
            vs = [plsc.load_gather(row_v, [base_v + j * L])
                  for j in range(NVC)]
            cand = jnp.where(vs[0] == M, iota, _BIG)
            for j in range(1, NVC):
                cand = jnp.minimum(cand,
                                   jnp.where(vs[j] == M, iota + j * L, _BIG))
            eloc = bfly(cand, jnp.minimum)
            ei = cstar * CHUNK + eloc
            store1(vals_vs[r], k_v, M)
            store1(idx_v, k_v + r * KK, ei)
            mm = jnp.where(iota == eloc, _NEG, vs[0])
            for j in range(1, NVC):
                mm = jnp.maximum(mm, jnp.where(iota + j * L == eloc,
                                               _NEG, vs[j]))
            newmax = bfly(mm, jnp.maximum)
            store1(row_v, ei + r * N, jnp.full((L,), _NEG, jnp.float32))
            for q in range(4):
                out.append(jnp.where(iota + q * L == cstar, newmax,
                                     carry[4 * r + q]))
        return tuple(out)
    lax.fori_loop(0, KK, step, carry0)

    for cp in cp1s:
        cp.wait()
    for r in range(2):
        for t in range(KK // L):
            iv = idx_v[pl.ds(r * KK + t * L, L)]
            g1_vs[r][pl.ds(t * L, L)] = plsc.load_gather(p1row_v, [iv + r * N])
            gidx_vs[r][pl.ds(t * L, L)] = iv + (rowA + r) * N
    cpgs = [pltpu.async_copy(p2_hbm.at[gidx_vs[r]], rows_vs[r], sem_g2)
            for r in range(2)]
    for r in range(2):
        pltpu.sync_copy(vals_vs[r], outw.at[rowA + r])
        pltpu.sync_copy(g1_vs[r], outg1.at[rowA + r])
    for r in range(2):
        cpgs[r].wait()
        pltpu.sync_copy(rows_vs[r], outg2.at[rowA + r])


def kernel(weights, prop1, prop2):
    p2 = prop2.reshape(R * N, D)
    outw, outg1, outg2 = _topk_gather(weights, prop1, p2)
    return (outw, outg1, outg2)


# speedup vs baseline: 1.0159x; 1.0062x over previous
import functools

import jax
import jax.numpy as jnp
from jax import lax
from jax.experimental import pallas as pl
from jax.experimental.pallas import tpu as pltpu
from jax.experimental.pallas import tpu_sc as plsc

R = 64
N = 8192
KK = 64
D = 128
L = 16
NCHUNK = 64
CHUNK = 128
NVC = CHUNK // L

_info = plsc.get_sparse_core_info()
NC, NS = _info.num_cores, _info.num_subcores
NW = NC * NS
ROWS_PER_W = R // NW

_NEG = float("-inf")
_BIG = 1 << 20

_mesh = plsc.VectorSubcoreMesh(core_axis_name="c", subcore_axis_name="s")


@functools.partial(
    pl.kernel,
    mesh=_mesh,
    out_type=[
        jax.ShapeDtypeStruct((R, KK), jnp.float32),
        jax.ShapeDtypeStruct((R, KK), jnp.float32),
        jax.ShapeDtypeStruct((R, KK, D), jnp.float32),
    ],
    scratch_types=[
        pltpu.VMEM((2 * N,), jnp.float32),
        pltpu.VMEM((2 * N,), jnp.float32),
        [pltpu.VMEM((KK,), jnp.float32) for _ in range(2)],
        [pltpu.VMEM((KK,), jnp.float32) for _ in range(2)],
        pltpu.VMEM((2 * KK,), jnp.int32),
        [pltpu.VMEM((KK,), jnp.int32) for _ in range(2)],
        [pltpu.VMEM((KK, D), jnp.float32) for _ in range(2)],
        pltpu.SemaphoreType.DMA,
        pltpu.SemaphoreType.DMA,
        pltpu.SemaphoreType.DMA,
    ],
    compiler_params=pltpu.CompilerParams(needs_layout_passes=False),
)
def _topk_gather(w_hbm, p1_hbm, p2_hbm, outw, outg1, outg2,
                 row_v, p1row_v, vals_vs, g1_vs, idx_v, gidx_vs,
                 rows_vs, sem_w, sem_p1, sem_g2):
    wid = lax.axis_index("s") * NC + lax.axis_index("c")
    rowA = wid * ROWS_PER_W
    iota = lax.iota(jnp.int32, L)
    lane0 = iota == 0

    def bfly(x, op):
        for s in (1, 2, 4, 8):
            x = op(x, x.at[jnp.bitwise_xor(iota, s)].get(
                mode="promise_in_bounds"))
        return x

    def bfly_argmax(v, i):
        for s in (1, 2, 4, 8):
            perm = jnp.bitwise_xor(iota, s)
            pv = v.at[perm].get(mode="promise_in_bounds")
            pi = i.at[perm].get(mode="promise_in_bounds")
            sw = (pv > v) | ((pv == v) & (pi < i))
            v = jnp.where(sw, pv, v)
            i = jnp.where(sw, pi, i)
        return v, i

    def store1(ref, pos_v, val_v):
        plsc.store_scatter(ref, [pos_v], val_v, mask=lane0)

    cps = [pltpu.async_copy(w_hbm.at[rowA + r], row_v.at[pl.ds(r * N, N)],
                            sem_w) for r in range(2)]
    cp1s = [pltpu.async_copy(p1_hbm.at[rowA + r],
                             p1row_v.at[pl.ds(r * N, N)], sem_p1)
            for r in range(2)]
    for cp in cps:
        cp.wait()

    base_idx = [[(iota + 16 * q) * CHUNK + r * N for q in range(4)]
                for r in range(2)]
    TU = 4

    def cm_step(t, carry):
        accs = list(carry)
        for u in range(TU):
            ph = (t * TU + u + iota) & (CHUNK - 1)
            p = 0
            for r in range(2):
                for q in range(4):
                    g = plsc.load_gather(row_v, [base_idx[r][q] + ph])
                    accs[p] = jnp.maximum(accs[p], g)
                    p += 1
        return tuple(accs)

    neg = jnp.full((L,), _NEG, jnp.float32)
    carry0 = lax.fori_loop(0, CHUNK // TU, cm_step, (neg,) * 8)

    def step(k, carry):
        out = []
        k_v = jnp.full((L,), k, jnp.int32)
        for r in range(2):
            c0, c1, c2, c3 = carry[4 * r:4 * r + 4]
            sa = c1 > c0
            va = jnp.where(sa, c1, c0)
            ia = jnp.where(sa, iota + L, iota)
            sb = c3 > c2
            vb = jnp.where(sb, c3, c2)
            ib = jnp.where(sb, iota + 3 * L, iota + 2 * L)
            sc = vb > va
            v4 = jnp.where(sc, vb, va)
            i4 = jnp.where(sc, ib, ia)
            M = bfly(v4, jnp.maximum)
            cstar = bfly(jnp.where(v4 == M, i4, _BIG), jnp.minimum)
            base_v = cstar * CHUNK + (r * N + iota)
            vs = [plsc.load_gather(row_v, [base_v + j * L])
                  for j in range(NVC)]
            cand = jnp.where(vs[0] == M, iota, _BIG)
            for j in range(1, NVC):
                cand = jnp.minimum(cand,
                                   jnp.where(vs[j] == M, iota + j * L, _BIG))
            eloc = bfly(cand, jnp.minimum)
            ei = cstar * CHUNK + eloc
            store1(vals_vs[r], k_v, M)
            store1(idx_v, k_v + r * KK, ei)
            below = jnp.where(vs[0] == M, _NEG, vs[0])
            cnt = jnp.where(vs[0] == M, 1, 0)
            for j in range(1, NVC):
                below = jnp.maximum(below,
                                    jnp.where(vs[j] == M, _NEG, vs[j]))
                cnt = cnt + jnp.where(vs[j] == M, 1, 0)
            mx_below = bfly(below, jnp.maximum)
            cnt = bfly(cnt, jnp.add)
            newmax = jnp.where(cnt >= 2, M, mx_below)
            store1(row_v, ei + r * N, jnp.full((L,), _NEG, jnp.float32))
            for q in range(4):
                out.append(jnp.where(iota + q * L == cstar, newmax,
                                     carry[4 * r + q]))
        return tuple(out)
    lax.fori_loop(0, KK, step, carry0)

    for cp in cp1s:
        cp.wait()
    for r in range(2):
        for t in range(KK // L):
            iv = idx_v[pl.ds(r * KK + t * L, L)]
            g1_vs[r][pl.ds(t * L, L)] = plsc.load_gather(p1row_v, [iv + r * N])
            gidx_vs[r][pl.ds(t * L, L)] = iv + (rowA + r) * N
    cpgs = [pltpu.async_copy(p2_hbm.at[gidx_vs[r]], rows_vs[r], sem_g2)
            for r in range(2)]
    for r in range(2):
        pltpu.sync_copy(vals_vs[r], outw.at[rowA + r])
        pltpu.sync_copy(g1_vs[r], outg1.at[rowA + r])
    for r in range(2):
        cpgs[r].wait()
        pltpu.sync_copy(rows_vs[r], outg2.at[rowA + r])


def kernel(weights, prop1, prop2):
    p2 = prop2.reshape(R * N, D)
    outw, outg1, outg2 = _topk_gather(weights, prop1, p2)
    return (outw, outg1, outg2)
